# Initial kernel scaffold; baseline (speedup 1.0000x reference)
import jax
import jax.numpy as jnp
from jax.experimental import pallas as pl


def kernel(noisy, deno, fflow, bflow):
    raise NotImplementedError("write your pallas kernel here")


# speedup vs baseline: 5.9443x; 5.9443x over previous
import numpy as np
import jax
import jax.numpy as jnp
from jax import lax
from jax.experimental import pallas as pl
from jax.experimental.pallas import tpu as pltpu
from jax.experimental.pallas import tpu_sc as plsc

WS = 9
PS = 7
K = 10
STRIDE0 = 4
T, C, H, W = 5, 3, 128, 128
PAD = PS // 2
HP = H + 2 * PAD
WP = W + 2 * PAD
FRAME = C * HP * WP
FRAME_PAD = 53888
NQ = H // STRIDE0
D = C * PS * PS
NWIN = WS * WS
NDT = 3
NG = 6
NGL = NG * 16
NEG = 10
DCOL = NDT * NGL
BIG = 1e30

_EOFF = [c * (HP * WP) + i * WP + j
         for c in range(C) for i in range(PS) for j in range(PS)]
_EOFF_PAD = np.array(_EOFF + [0] * (NEG * 16 - D), dtype=np.int32)

_OWI = np.array([n // WS - WS // 2 if n < NWIN else 0 for n in range(NGL)],
                dtype=np.int32)
_OWJ = np.array([n % WS - WS // 2 if n < NWIN else 0 for n in range(NGL)],
                dtype=np.int32)

_NC = 2
_NS = 16
_NW = _NC * _NS


def _sc_body(vp_h, cbase_h, eoff_h, out_h,
             fbuf, qpat, dbuf, cbbuf, eofb, accb):
    wid = lax.axis_index("s") * _NC + lax.axis_index("c")
    pltpu.sync_copy(eoff_h, eofb)
    accb[...] = jnp.zeros((16,), jnp.float32)

    ybase = wid * STRIDE0 * WP

    def t_body(t, tcarry):
        pltpu.sync_copy(cbase_h.at[wid, t], cbbuf)
        pltpu.sync_copy(vp_h.at[t], fbuf)

        def qg_body(q, c):
            qb = ybase + q * STRIDE0

            def g_body(g, c2):
                ev = eofb[pl.ds(g * 16, 16)]
                pix = plsc.load_gather(fbuf, [ev + qb])
                qpat[q, pl.ds(g * 16, 16)] = pix
                return c2
            return lax.fori_loop(0, NEG, g_body, c)
        lax.fori_loop(0, NQ, qg_body, 0)

        def dt_body(dt, c):
            tf = jnp.clip(t + dt - 1, 0, T - 1)
            pltpu.sync_copy(vp_h.at[tf], fbuf)

            def q_body(q, c2):
                def g_body(g, c3):
                    gb = g * 16
                    cb = cbbuf[pl.ds((dt * NQ + q) * NGL + gb, 16)]
                    acc = jnp.zeros((16,), jnp.float32)
                    for eg in range(NEG):
                        qv = qpat[q, pl.ds(eg * 16, 16)]
                        for l in range(16):
                            e = eg * 16 + l
                            if e >= D:
                                break
                            pix = plsc.load_gather(fbuf, [cb + _EOFF[e]])
                            d = qv[l] - pix
                            acc = acc + d * d
                    n = lax.broadcasted_iota(jnp.int32, (16,), 0) + gb
                    acc = jnp.where(n < NWIN, acc, jnp.float32(BIG))
                    dbuf[q, pl.ds(dt * NGL + gb, 16)] = acc
                    return c3
                return lax.fori_loop(0, NG, g_body, c2)
            return lax.fori_loop(0, NQ, q_body, c)
        lax.fori_loop(0, NDT, dt_body, 0)

        def qt_body(q, c):
            def m_body(i, r):
                vs = lax.sort(dbuf[q, pl.ds(i * 16, 16)])
                return lax.sort(jnp.minimum(r, lax.rev(vs, (0,))))
            r0 = lax.sort(dbuf[q, pl.ds(0, 16)])
            r = lax.fori_loop(1, NDT * NG, m_body, r0)
            lane = lax.broadcasted_iota(jnp.int32, (16,), 0)
            contrib = jnp.where(lane < K, r, jnp.float32(0.0))
            accb[...] = accb[...] + contrib
            return c
        lax.fori_loop(0, NQ, qt_body, 0)
        return tcarry

    lax.fori_loop(0, T, t_body, 0)
    pltpu.sync_copy(accb, out_h.at[wid])


@jax.jit
def kernel(noisy, deno, fflow, bflow):
    del deno
    vid = noisy[0]
    vp = jnp.pad(vid, ((0, 0), (0, 0), (PAD, PAD), (PAD, PAD)), mode='edge')
    vp = vp.reshape(T, FRAME)
    vp = jnp.pad(vp, ((0, 0), (0, FRAME_PAD - FRAME)))

    qh = jnp.arange(0, H, STRIDE0, dtype=jnp.float32)
    owi = jnp.asarray(_OWI)
    owj = jnp.asarray(_OWJ)
    cbs = []
    for dtv in (-1, 0, 1):
        if dtv == 0:
            fh = jnp.zeros((T, NQ, NQ), jnp.float32)
            fw = fh
        elif dtv > 0:
            fw = fflow[0, :, 0, ::STRIDE0, ::STRIDE0] * dtv
            fh = fflow[0, :, 1, ::STRIDE0, ::STRIDE0] * dtv
        else:
            fw = bflow[0, :, 0, ::STRIDE0, ::STRIDE0] * (-dtv)
            fh = bflow[0, :, 1, ::STRIDE0, ::STRIDE0] * (-dtv)
        c0h = jnp.round(qh[None, :, None] + fh).astype(jnp.int32)
        c0w = jnp.round(qh[None, None, :] + fw).astype(jnp.int32)
        u = jnp.clip(c0h[..., None] + owi[None, None, None, :], 0, H - 1)
        v = jnp.clip(c0w[..., None] + owj[None, None, None, :], 0, W - 1)
        cbs.append(u * WP + v)
    cb = jnp.stack(cbs, axis=1)
    cb_r = jnp.transpose(cb, (2, 0, 1, 3, 4)).reshape(NQ, T, NDT * NQ * NGL)

    eoff = jnp.asarray(_EOFF_PAD)

    mesh = plsc.VectorSubcoreMesh(core_axis_name="c", subcore_axis_name="s")
    run = pl.kernel(
        _sc_body,
        out_type=jax.ShapeDtypeStruct((_NW, 16), jnp.float32),
        mesh=mesh,
        compiler_params=pltpu.CompilerParams(needs_layout_passes=False),
        scratch_types=[
            pltpu.VMEM((FRAME_PAD,), jnp.float32),
            pltpu.VMEM((NQ, NEG * 16), jnp.float32),
            pltpu.VMEM((NQ, DCOL), jnp.float32),
            pltpu.VMEM((NDT * NQ * NGL,), jnp.int32),
            pltpu.VMEM((NEG * 16,), jnp.int32),
            pltpu.VMEM((16,), jnp.float32),
        ],
    )
    partials = run(vp, cb_r, eoff)
    return jnp.sum(partials) / jnp.float32(T * NQ * NQ * K)


# speedup vs baseline: 6.2039x; 1.0437x over previous
import numpy as np
import jax
import jax.numpy as jnp
from jax import lax
from jax.experimental import pallas as pl
from jax.experimental.pallas import tpu as pltpu
from jax.experimental.pallas import tpu_sc as plsc

WS = 9
PS = 7
K = 10
STRIDE0 = 4
T, C, H, W = 5, 3, 128, 128
PAD = PS // 2
HP = H + 2 * PAD
WP = W + 2 * PAD
FRAME = C * HP * WP
FRAME_PAD = 53888
NQ = H // STRIDE0
D = C * PS * PS
NWIN = WS * WS
NDT = 3
NG = 6
NGL = NG * 16
NEG = 10
DCOL = NDT * NGL
BIG = 1e30

_EOFF = [c * (HP * WP) + i * WP + j
         for c in range(C) for i in range(PS) for j in range(PS)]
_EOFF_PAD = np.array(_EOFF + [0] * (NEG * 16 - D), dtype=np.int32)

_OWI = np.array([n // WS - WS // 2 if n < NWIN else 0 for n in range(NGL)],
                dtype=np.int32)
_OWJ = np.array([n % WS - WS // 2 if n < NWIN else 0 for n in range(NGL)],
                dtype=np.int32)

_NC = 2
_NS = 16
_NW = _NC * _NS


def _sc_body(vp_h, cbase_h, eoff_h, out_h,
             fba, fbb, qpat, rbuf, cbbuf, eofb, accb, dsem):
    wid = lax.axis_index("s") * _NC + lax.axis_index("c")
    pltpu.sync_copy(eoff_h, eofb)
    accb[...] = jnp.zeros((16,), jnp.float32)
    pltpu.sync_copy(vp_h.at[0], fba)

    ybase = wid * STRIDE0 * WP

    def compute_dt(dt, fbuf):
        def q_body(q, c2):
            def g_body(g, c3):
                gb = g * 16
                cb = cbbuf[pl.ds((dt * NQ + q) * NGL + gb, 16)]
                acc = jnp.zeros((16,), jnp.float32)
                for eg in range(NEG):
                    qv = qpat[q, pl.ds(eg * 16, 16)]
                    for l in range(16):
                        e = eg * 16 + l
                        if e >= D:
                            break
                        pix = plsc.load_gather(fbuf, [cb + _EOFF[e]])
                        d = qv[l] - pix
                        acc = acc + d * d
                n = lax.broadcasted_iota(jnp.int32, (16,), 0) + gb
                acc = jnp.where(n < NWIN, acc, jnp.float32(BIG))
                r = rbuf[q, :]
                rbuf[q, :] = lax.sort(
                    jnp.minimum(r, lax.rev(lax.sort(acc), (0,))))
                return c3
            return lax.fori_loop(0, NG, g_body, c2)
        lax.fori_loop(0, NQ, q_body, 0)

    def t_body(t, tcarry):
        pltpu.sync_copy(cbase_h.at[wid, t], cbbuf)

        def qg_body(q, c):
            qb = ybase + q * STRIDE0

            def g_body(g, c2):
                ev = eofb[pl.ds(g * 16, 16)]
                pix = plsc.load_gather(fba, [ev + qb])
                qpat[q, pl.ds(g * 16, 16)] = pix
                return c2
            return lax.fori_loop(0, NEG, g_body, c)
        lax.fori_loop(0, NQ, qg_body, 0)

        def ri_body(q, c):
            rbuf[q, :] = jnp.full((16,), BIG, jnp.float32)
            return c
        lax.fori_loop(0, NQ, ri_body, 0)

        cpB = pltpu.async_copy(vp_h.at[jnp.clip(t - 1, 0, T - 1)], fbb, dsem)
        compute_dt(1, fba)
        cpB.wait()
        cpA = pltpu.async_copy(vp_h.at[jnp.clip(t + 1, 0, T - 1)], fba, dsem)
        compute_dt(0, fbb)
        cpA.wait()
        compute_dt(2, fba)

        def qt_body(q, c):
            r = rbuf[q, :]
            lane = lax.broadcasted_iota(jnp.int32, (16,), 0)
            accb[...] = accb[...] + jnp.where(lane < K, r, jnp.float32(0.0))
            return c
        lax.fori_loop(0, NQ, qt_body, 0)
        return tcarry

    lax.fori_loop(0, T, t_body, 0)
    pltpu.sync_copy(accb, out_h.at[wid])


@jax.jit
def kernel(noisy, deno, fflow, bflow):
    del deno
    vid = noisy[0]
    vp = jnp.pad(vid, ((0, 0), (0, 0), (PAD, PAD), (PAD, PAD)), mode='edge')
    vp = vp.reshape(T, FRAME)
    vp = jnp.pad(vp, ((0, 0), (0, FRAME_PAD - FRAME)))

    qh = jnp.arange(0, H, STRIDE0, dtype=jnp.float32)
    owi = jnp.asarray(_OWI)
    owj = jnp.asarray(_OWJ)
    cbs = []
    for dtv in (-1, 0, 1):
        if dtv == 0:
            fh = jnp.zeros((T, NQ, NQ), jnp.float32)
            fw = fh
        elif dtv > 0:
            fw = fflow[0, :, 0, ::STRIDE0, ::STRIDE0] * dtv
            fh = fflow[0, :, 1, ::STRIDE0, ::STRIDE0] * dtv
        else:
            fw = bflow[0, :, 0, ::STRIDE0, ::STRIDE0] * (-dtv)
            fh = bflow[0, :, 1, ::STRIDE0, ::STRIDE0] * (-dtv)
        c0h = jnp.round(qh[None, :, None] + fh).astype(jnp.int32)
        c0w = jnp.round(qh[None, None, :] + fw).astype(jnp.int32)
        u = jnp.clip(c0h[..., None] + owi[None, None, None, :], 0, H - 1)
        v = jnp.clip(c0w[..., None] + owj[None, None, None, :], 0, W - 1)
        cbs.append(u * WP + v)
    cb = jnp.stack(cbs, axis=1)
    cb_r = jnp.transpose(cb, (2, 0, 1, 3, 4)).reshape(NQ, T, NDT * NQ * NGL)

    eoff = jnp.asarray(_EOFF_PAD)

    mesh = plsc.VectorSubcoreMesh(core_axis_name="c", subcore_axis_name="s")
    run = pl.kernel(
        _sc_body,
        out_type=jax.ShapeDtypeStruct((_NW, 16), jnp.float32),
        mesh=mesh,
        compiler_params=pltpu.CompilerParams(needs_layout_passes=False),
        scratch_types=[
            pltpu.VMEM((FRAME_PAD,), jnp.float32),
            pltpu.VMEM((FRAME_PAD,), jnp.float32),
            pltpu.VMEM((NQ, NEG * 16), jnp.float32),
            pltpu.VMEM((NQ, 16), jnp.float32),
            pltpu.VMEM((NDT * NQ * NGL,), jnp.int32),
            pltpu.VMEM((NEG * 16,), jnp.int32),
            pltpu.VMEM((16,), jnp.float32),
            pltpu.SemaphoreType.DMA,
        ],
    )
    partials = run(vp, cb_r, eoff)
    return jnp.sum(partials) / jnp.float32(T * NQ * NQ * K)


# speedup vs baseline: 7.2303x; 1.1654x over previous
import numpy as np
import jax
import jax.numpy as jnp
from jax import lax
from jax.experimental import pallas as pl
from jax.experimental.pallas import tpu as pltpu
from jax.experimental.pallas import tpu_sc as plsc

WS = 9
PS = 7
K = 10
STRIDE0 = 4
T, C, H, W = 5, 3, 128, 128
PAD = PS // 2
HP = H + 2 * PAD
WP = W + 2 * PAD
FRAME = C * HP * WP
FRAME_PAD = 53888
NQ = H // STRIDE0
D = C * PS * PS
NWIN = WS * WS
NDT = 3
NG = 6
NGL = NG * 16
NEG = 10
DCOL = NDT * NGL
BIG = 1e30

_EOFF = [c * (HP * WP) + i * WP + j
         for c in range(C) for i in range(PS) for j in range(PS)]
_EOFF_PAD = np.array(_EOFF + [0] * (NEG * 16 - D), dtype=np.int32)

_OWI = np.array([n // WS - WS // 2 if n < NWIN else 0 for n in range(NGL)],
                dtype=np.int32)
_OWJ = np.array([n % WS - WS // 2 if n < NWIN else 0 for n in range(NGL)],
                dtype=np.int32)

_NC = 2
_NS = 16
_NW = _NC * _NS


def _sc_body(vp_h, cbase_h, eoff_h, out_h,
             fba, fbb, qpat, rbuf, cbbuf, eofb, accb, dsem):
    wid = lax.axis_index("s") * _NC + lax.axis_index("c")
    pltpu.sync_copy(eoff_h, eofb)
    accb[...] = jnp.zeros((16,), jnp.float32)
    pltpu.sync_copy(vp_h.at[0], fba)

    ybase = wid * STRIDE0 * WP

    def compute_dt(dt, fbuf):
        def q_body(q, c2):
            def g_body(g, c3):
                gb = g * 16
                cb = cbbuf[pl.ds((dt * NQ + q) * NGL + gb, 16)]
                accs = [jnp.zeros((16,), jnp.float32) for _ in range(8)]
                for eg in range(NEG):
                    qv = qpat[q, pl.ds(eg * 16, 16)]
                    for l in range(16):
                        e = eg * 16 + l
                        if e >= D:
                            break
                        pix = plsc.load_gather(fbuf, [cb + _EOFF[e]])
                        d = qv[l] - pix
                        accs[e % 8] = accs[e % 8] + d * d
                acc = ((accs[0] + accs[1]) + (accs[2] + accs[3])) + (
                    (accs[4] + accs[5]) + (accs[6] + accs[7]))
                n = lax.broadcasted_iota(jnp.int32, (16,), 0) + gb
                acc = jnp.where(n < NWIN, acc, jnp.float32(BIG))
                r = rbuf[q, :]
                rbuf[q, :] = lax.sort(
                    jnp.minimum(r, lax.rev(lax.sort(acc), (0,))))
                return c3
            return lax.fori_loop(0, NG, g_body, c2)
        lax.fori_loop(0, NQ, q_body, 0)

    def t_body(t, tcarry):
        pltpu.sync_copy(cbase_h.at[wid, t], cbbuf)

        def qg_body(q, c):
            qb = ybase + q * STRIDE0

            def g_body(g, c2):
                ev = eofb[pl.ds(g * 16, 16)]
                pix = plsc.load_gather(fba, [ev + qb])
                qpat[q, pl.ds(g * 16, 16)] = pix
                return c2
            return lax.fori_loop(0, NEG, g_body, c)
        lax.fori_loop(0, NQ, qg_body, 0)

        def ri_body(q, c):
            rbuf[q, :] = jnp.full((16,), BIG, jnp.float32)
            return c
        lax.fori_loop(0, NQ, ri_body, 0)

        cpB = pltpu.async_copy(vp_h.at[jnp.clip(t - 1, 0, T - 1)], fbb, dsem)
        compute_dt(1, fba)
        cpB.wait()
        cpA = pltpu.async_copy(vp_h.at[jnp.clip(t + 1, 0, T - 1)], fba, dsem)
        compute_dt(0, fbb)
        cpA.wait()
        compute_dt(2, fba)

        def qt_body(q, c):
            r = rbuf[q, :]
            lane = lax.broadcasted_iota(jnp.int32, (16,), 0)
            accb[...] = accb[...] + jnp.where(lane < K, r, jnp.float32(0.0))
            return c
        lax.fori_loop(0, NQ, qt_body, 0)
        return tcarry

    lax.fori_loop(0, T, t_body, 0)
    pltpu.sync_copy(accb, out_h.at[wid])


@jax.jit
def kernel(noisy, deno, fflow, bflow):
    del deno
    vid = noisy[0]
    vp = jnp.pad(vid, ((0, 0), (0, 0), (PAD, PAD), (PAD, PAD)), mode='edge')
    vp = vp.reshape(T, FRAME)
    vp = jnp.pad(vp, ((0, 0), (0, FRAME_PAD - FRAME)))

    qh = jnp.arange(0, H, STRIDE0, dtype=jnp.float32)
    owi = jnp.asarray(_OWI)
    owj = jnp.asarray(_OWJ)
    cbs = []
    for dtv in (-1, 0, 1):
        if dtv == 0:
            fh = jnp.zeros((T, NQ, NQ), jnp.float32)
            fw = fh
        elif dtv > 0:
            fw = fflow[0, :, 0, ::STRIDE0, ::STRIDE0] * dtv
            fh = fflow[0, :, 1, ::STRIDE0, ::STRIDE0] * dtv
        else:
            fw = bflow[0, :, 0, ::STRIDE0, ::STRIDE0] * (-dtv)
            fh = bflow[0, :, 1, ::STRIDE0, ::STRIDE0] * (-dtv)
        c0h = jnp.round(qh[None, :, None] + fh).astype(jnp.int32)
        c0w = jnp.round(qh[None, None, :] + fw).astype(jnp.int32)
        u = jnp.clip(c0h[..., None] + owi[None, None, None, :], 0, H - 1)
        v = jnp.clip(c0w[..., None] + owj[None, None, None, :], 0, W - 1)
        cbs.append(u * WP + v)
    cb = jnp.stack(cbs, axis=1)
    cb_r = jnp.transpose(cb, (2, 0, 1, 3, 4)).reshape(NQ, T, NDT * NQ * NGL)

    eoff = jnp.asarray(_EOFF_PAD)

    mesh = plsc.VectorSubcoreMesh(core_axis_name="c", subcore_axis_name="s")
    run = pl.kernel(
        _sc_body,
        out_type=jax.ShapeDtypeStruct((_NW, 16), jnp.float32),
        mesh=mesh,
        compiler_params=pltpu.CompilerParams(needs_layout_passes=False),
        scratch_types=[
            pltpu.VMEM((FRAME_PAD,), jnp.float32),
            pltpu.VMEM((FRAME_PAD,), jnp.float32),
            pltpu.VMEM((NQ, NEG * 16), jnp.float32),
            pltpu.VMEM((NQ, 16), jnp.float32),
            pltpu.VMEM((NDT * NQ * NGL,), jnp.int32),
            pltpu.VMEM((NEG * 16,), jnp.int32),
            pltpu.VMEM((16,), jnp.float32),
            pltpu.SemaphoreType.DMA,
        ],
    )
    partials = run(vp, cb_r, eoff)
    return jnp.sum(partials) / jnp.float32(T * NQ * NQ * K)


# speedup vs baseline: 7.8805x; 1.0899x over previous
import numpy as np
import jax
import jax.numpy as jnp
from jax import lax
from jax.experimental import pallas as pl
from jax.experimental.pallas import tpu as pltpu
from jax.experimental.pallas import tpu_sc as plsc

WS = 9
PS = 7
K = 10
STRIDE0 = 4
T, C, H, W = 5, 3, 128, 128
PAD = PS // 2
HP = H + 2 * PAD
WP = W + 2 * PAD
FRAME = C * HP * WP
FRAME_PAD = 53888
NQ = H // STRIDE0
D = C * PS * PS
NWIN = WS * WS
NDT = 3
NG = 6
NGL = NG * 16
NEG = 10
DCOL = NDT * NGL
BIG = 1e30

_EOFF = [c * (HP * WP) + i * WP + j
         for c in range(C) for i in range(PS) for j in range(PS)]
_EOFF_PAD = np.array(_EOFF + [0] * (NEG * 16 - D), dtype=np.int32)

_OWI = np.array([n // WS - WS // 2 if n < NWIN else 0 for n in range(NGL)],
                dtype=np.int32)
_OWJ = np.array([n % WS - WS // 2 if n < NWIN else 0 for n in range(NGL)],
                dtype=np.int32)

_NC = 2
_NS = 16
_NW = _NC * _NS


def _sc_body(vp_h, cbase_h, eoff_h, out_h,
             fba, fbb, qpat, qpatT, rbuf, lbuf, cbbuf, eofb, accb, dsem):
    wid = lax.axis_index("s") * _NC + lax.axis_index("c")
    pltpu.sync_copy(eoff_h, eofb)
    accb[...] = jnp.zeros((16,), jnp.float32)
    pltpu.sync_copy(vp_h.at[0], fba)

    ybase = wid * STRIDE0 * WP

    def compute_dt(t, dt, fbuf):
        pltpu.sync_copy(cbase_h.at[(wid * T + t) * NDT + dt], cbbuf)
        def q_body(q, c2):
            def g_body(g, c3):
                gb = g * 16
                cb = cbbuf[pl.ds(q * NGL + gb, 16)]
                accs = [jnp.zeros((16,), jnp.float32) for _ in range(8)]
                for eg in range(NEG):
                    qv = qpat[q, pl.ds(eg * 16, 16)]
                    for l in range(16):
                        e = eg * 16 + l
                        if e >= D:
                            break
                        pix = plsc.load_gather(fbuf, [cb + _EOFF[e]])
                        d = qv[l] - pix
                        accs[e % 8] = accs[e % 8] + d * d
                acc = ((accs[0] + accs[1]) + (accs[2] + accs[3])) + (
                    (accs[4] + accs[5]) + (accs[6] + accs[7]))
                r = rbuf[q, :]
                rbuf[q, :] = lax.sort(
                    jnp.minimum(r, lax.rev(lax.sort(acc), (0,))))
                return c3
            return lax.fori_loop(0, NG - 1, g_body, c2)
        lax.fori_loop(0, NQ, q_body, 0)

        def lo_body(qg, c2):
            lane = lax.broadcasted_iota(jnp.int32, (16,), 0)
            qlane = lane + qg * 16
            cb = plsc.load_gather(cbbuf, [qlane * NGL + (NWIN - 1)])

            def le_body(eg, accs):
                ev = eofb[pl.ds(eg * 16, 16)]
                base = eg * 16 * NQ + qg * 16
                nacc = list(accs)
                for l in range(16):
                    pix = plsc.load_gather(fbuf, [cb + ev[l]])
                    qvv = qpatT[pl.ds(base + l * NQ, 16)]
                    d = qvv - pix
                    nacc[l % 8] = nacc[l % 8] + d * d
                return tuple(nacc)
            accs = lax.fori_loop(
                0, NEG - 1, le_body,
                tuple(jnp.zeros((16,), jnp.float32) for _ in range(8)))
            ev = eofb[pl.ds((NEG - 1) * 16, 16)]
            base = (NEG - 1) * 16 * NQ + qg * 16
            accs = list(accs)
            for l in range(D - (NEG - 1) * 16):
                pix = plsc.load_gather(fbuf, [cb + ev[l]])
                qvv = qpatT[pl.ds(base + l * NQ, 16)]
                d = qvv - pix
                accs[l % 8] = accs[l % 8] + d * d
            acc = ((accs[0] + accs[1]) + (accs[2] + accs[3])) + (
                (accs[4] + accs[5]) + (accs[6] + accs[7]))
            plsc.store_scatter(lbuf, [qlane * 16 + dt], acc)
            return c2
        lax.fori_loop(0, 2, lo_body, 0)

    def t_body(t, tcarry):

        def qg_body(q, c):
            qb = ybase + q * STRIDE0

            def g_body(g, c2):
                ev = eofb[pl.ds(g * 16, 16)]
                pix = plsc.load_gather(fba, [ev + qb])
                qpat[q, pl.ds(g * 16, 16)] = pix
                lane = lax.broadcasted_iota(jnp.int32, (16,), 0)
                plsc.store_scatter(qpatT, [(lane + g * 16) * NQ + q], pix)
                return c2
            return lax.fori_loop(0, NEG, g_body, c)
        lax.fori_loop(0, NQ, qg_body, 0)

        def ri_body(q, c):
            rbuf[q, :] = jnp.full((16,), BIG, jnp.float32)
            lbuf[pl.ds(q * 16, 16)] = jnp.full((16,), BIG, jnp.float32)
            return c
        lax.fori_loop(0, NQ, ri_body, 0)

        cpB = pltpu.async_copy(vp_h.at[jnp.clip(t - 1, 0, T - 1)], fbb, dsem)
        compute_dt(t, 1, fba)
        cpB.wait()
        cpA = pltpu.async_copy(vp_h.at[jnp.clip(t + 1, 0, T - 1)], fba, dsem)
        compute_dt(t, 0, fbb)
        cpA.wait()
        compute_dt(t, 2, fba)

        def qt_body(q, c):
            r = rbuf[q, :]
            lv = lbuf[pl.ds(q * 16, 16)]
            r = lax.sort(jnp.minimum(r, lax.rev(lax.sort(lv), (0,))))
            lane = lax.broadcasted_iota(jnp.int32, (16,), 0)
            accb[...] = accb[...] + jnp.where(lane < K, r, jnp.float32(0.0))
            return c
        lax.fori_loop(0, NQ, qt_body, 0)
        return tcarry

    lax.fori_loop(0, T, t_body, 0)
    pltpu.sync_copy(accb, out_h.at[wid])


@jax.jit
def kernel(noisy, deno, fflow, bflow):
    del deno
    vid = noisy[0]
    vp = jnp.pad(vid, ((0, 0), (0, 0), (PAD, PAD), (PAD, PAD)), mode='edge')
    vp = vp.reshape(T, FRAME)
    vp = jnp.pad(vp, ((0, 0), (0, FRAME_PAD - FRAME)))

    qh = jnp.arange(0, H, STRIDE0, dtype=jnp.float32)
    owi = jnp.asarray(_OWI)
    owj = jnp.asarray(_OWJ)
    cbs = []
    for dtv in (-1, 0, 1):
        if dtv == 0:
            fh = jnp.zeros((T, NQ, NQ), jnp.float32)
            fw = fh
        elif dtv > 0:
            fw = fflow[0, :, 0, ::STRIDE0, ::STRIDE0] * dtv
            fh = fflow[0, :, 1, ::STRIDE0, ::STRIDE0] * dtv
        else:
            fw = bflow[0, :, 0, ::STRIDE0, ::STRIDE0] * (-dtv)
            fh = bflow[0, :, 1, ::STRIDE0, ::STRIDE0] * (-dtv)
        c0h = jnp.round(qh[None, :, None] + fh).astype(jnp.int32)
        c0w = jnp.round(qh[None, None, :] + fw).astype(jnp.int32)
        u = jnp.clip(c0h[..., None] + owi[None, None, None, :], 0, H - 1)
        v = jnp.clip(c0w[..., None] + owj[None, None, None, :], 0, W - 1)
        cbs.append(u * WP + v)
    cb = jnp.stack(cbs, axis=1)
    cb_r = jnp.transpose(cb, (2, 0, 1, 3, 4)).reshape(NQ * T * NDT, NQ * NGL)

    eoff = jnp.asarray(_EOFF_PAD)

    mesh = plsc.VectorSubcoreMesh(core_axis_name="c", subcore_axis_name="s")
    run = pl.kernel(
        _sc_body,
        out_type=jax.ShapeDtypeStruct((_NW, 16), jnp.float32),
        mesh=mesh,
        compiler_params=pltpu.CompilerParams(needs_layout_passes=False),
        scratch_types=[
            pltpu.VMEM((FRAME_PAD,), jnp.float32),
            pltpu.VMEM((FRAME_PAD,), jnp.float32),
            pltpu.VMEM((NQ, NEG * 16), jnp.float32),
            pltpu.VMEM((NEG * 16 * NQ,), jnp.float32),
            pltpu.VMEM((NQ, 16), jnp.float32),
            pltpu.VMEM((NQ * 16,), jnp.float32),
            pltpu.VMEM((NQ * NGL,), jnp.int32),
            pltpu.VMEM((NEG * 16,), jnp.int32),
            pltpu.VMEM((16,), jnp.float32),
            pltpu.SemaphoreType.DMA,
        ],
    )
    partials = run(vp, cb_r, eoff)
    return jnp.sum(partials) / jnp.float32(T * NQ * NQ * K)


# speedup vs baseline: 9.2246x; 1.1706x over previous
import numpy as np
import jax
import jax.numpy as jnp
from jax import lax
from jax.experimental import pallas as pl
from jax.experimental.pallas import tpu as pltpu
from jax.experimental.pallas import tpu_sc as plsc

WS = 9
PS = 7
K = 10
STRIDE0 = 4
T, C, H, W = 5, 3, 128, 128
PAD = PS // 2
HP = H + 2 * PAD
WP = 136
FRAME_PAD = C * HP * WP
NQ = H // STRIDE0
D = C * PS * PS
NWIN = WS * WS
NDT = 3
NG = 6
NGL = NG * 16
NEG = 10
DCOL = NDT * NGL
BIG = 1e30

_EOFF = [c * (HP * WP) + i * WP + j
         for c in range(C) for i in range(PS) for j in range(PS)]
_EOFF_PAD = np.array(_EOFF + [0] * (NEG * 16 - D), dtype=np.int32)

_OWI = np.array([n // WS - WS // 2 if n < NWIN else 0 for n in range(NGL)],
                dtype=np.int32)
_OWJ = np.array([n % WS - WS // 2 if n < NWIN else 0 for n in range(NGL)],
                dtype=np.int32)

_NC = 2
_NS = 16
_NW = _NC * _NS


def _sc_body(vp_h, cbase_h, eoff_h, out_h,
             fba, fbb, qpat, qpatT, rbuf, lbuf, cbbuf, eofb, accb, dsem):
    wid = lax.axis_index("s") * _NC + lax.axis_index("c")
    pltpu.sync_copy(eoff_h, eofb)
    accb[...] = jnp.zeros((16,), jnp.float32)
    pltpu.sync_copy(vp_h.at[0], fba)

    ybase = wid * STRIDE0 * WP

    def compute_dt(t, dt, fbuf):
        pltpu.sync_copy(cbase_h.at[(wid * T + t) * NDT + dt], cbbuf)
        def q_body(q, c2):
            def g_body(g, c3):
                gb = g * 16
                cb = cbbuf[pl.ds(q * NGL + gb, 16)]
                cbj = [cb + j for j in range(PS)]
                accs = [jnp.zeros((16,), jnp.float32) for _ in range(8)]
                qv = None
                for e in range(D):
                    if e % 16 == 0:
                        qv = qpat[q, pl.ds(e, 16)]
                    ci, j = divmod(e, PS)
                    rb = (ci // PS) * (HP * WP) + (ci % PS) * WP
                    fv = fbuf.at[pl.ds(rb, FRAME_PAD - rb)]
                    pix = plsc.load_gather(fv, [cbj[j]])
                    d = qv[e % 16] - pix
                    accs[e % 8] = accs[e % 8] + d * d
                acc = ((accs[0] + accs[1]) + (accs[2] + accs[3])) + (
                    (accs[4] + accs[5]) + (accs[6] + accs[7]))
                r = rbuf[q, :]
                rbuf[q, :] = lax.sort(
                    jnp.minimum(r, lax.rev(lax.sort(acc), (0,))))
                return c3
            return lax.fori_loop(0, NG - 1, g_body, c2)
        lax.fori_loop(0, NQ, q_body, 0)

        def lo_body(qg, c2):
            lane = lax.broadcasted_iota(jnp.int32, (16,), 0)
            qlane = lane + qg * 16
            cb = plsc.load_gather(cbbuf, [qlane * NGL + (NWIN - 1)])

            def le_body(eg, accs):
                ev = eofb[pl.ds(eg * 16, 16)]
                base = eg * 16 * NQ + qg * 16
                nacc = list(accs)
                for l in range(16):
                    pix = plsc.load_gather(fbuf, [cb + ev[l]])
                    qvv = qpatT[pl.ds(base + l * NQ, 16)]
                    d = qvv - pix
                    nacc[l % 8] = nacc[l % 8] + d * d
                return tuple(nacc)
            accs = lax.fori_loop(
                0, NEG - 1, le_body,
                tuple(jnp.zeros((16,), jnp.float32) for _ in range(8)))
            ev = eofb[pl.ds((NEG - 1) * 16, 16)]
            base = (NEG - 1) * 16 * NQ + qg * 16
            accs = list(accs)
            for l in range(D - (NEG - 1) * 16):
                pix = plsc.load_gather(fbuf, [cb + ev[l]])
                qvv = qpatT[pl.ds(base + l * NQ, 16)]
                d = qvv - pix
                accs[l % 8] = accs[l % 8] + d * d
            acc = ((accs[0] + accs[1]) + (accs[2] + accs[3])) + (
                (accs[4] + accs[5]) + (accs[6] + accs[7]))
            plsc.store_scatter(lbuf, [qlane * 16 + dt], acc)
            return c2
        lax.fori_loop(0, 2, lo_body, 0)

    def t_body(t, tcarry):

        def qg_body(q, c):
            qb = ybase + q * STRIDE0

            def g_body(g, c2):
                ev = eofb[pl.ds(g * 16, 16)]
                pix = plsc.load_gather(fba, [ev + qb])
                qpat[q, pl.ds(g * 16, 16)] = pix
                lane = lax.broadcasted_iota(jnp.int32, (16,), 0)
                plsc.store_scatter(qpatT, [(lane + g * 16) * NQ + q], pix)
                return c2
            return lax.fori_loop(0, NEG, g_body, c)
        lax.fori_loop(0, NQ, qg_body, 0)

        def ri_body(q, c):
            rbuf[q, :] = jnp.full((16,), BIG, jnp.float32)
            lbuf[pl.ds(q * 16, 16)] = jnp.full((16,), BIG, jnp.float32)
            return c
        lax.fori_loop(0, NQ, ri_body, 0)

        cpB = pltpu.async_copy(vp_h.at[jnp.clip(t - 1, 0, T - 1)], fbb, dsem)
        compute_dt(t, 1, fba)
        cpB.wait()
        cpA = pltpu.async_copy(vp_h.at[jnp.clip(t + 1, 0, T - 1)], fba, dsem)
        compute_dt(t, 0, fbb)
        cpA.wait()
        compute_dt(t, 2, fba)

        def qt_body(q, c):
            r = rbuf[q, :]
            lv = lbuf[pl.ds(q * 16, 16)]
            r = lax.sort(jnp.minimum(r, lax.rev(lax.sort(lv), (0,))))
            lane = lax.broadcasted_iota(jnp.int32, (16,), 0)
            accb[...] = accb[...] + jnp.where(lane < K, r, jnp.float32(0.0))
            return c
        lax.fori_loop(0, NQ, qt_body, 0)
        return tcarry

    lax.fori_loop(0, T, t_body, 0)
    pltpu.sync_copy(accb, out_h.at[wid])


@jax.jit
def kernel(noisy, deno, fflow, bflow):
    del deno
    vid = noisy[0]
    vp = jnp.pad(vid, ((0, 0), (0, 0), (PAD, PAD), (PAD, WP - W - PAD)),
                 mode='edge')
    vp = vp.reshape(T, FRAME_PAD)

    qh = jnp.arange(0, H, STRIDE0, dtype=jnp.float32)
    owi = jnp.asarray(_OWI)
    owj = jnp.asarray(_OWJ)
    cbs = []
    for dtv in (-1, 0, 1):
        if dtv == 0:
            fh = jnp.zeros((T, NQ, NQ), jnp.float32)
            fw = fh
        elif dtv > 0:
            fw = fflow[0, :, 0, ::STRIDE0, ::STRIDE0] * dtv
            fh = fflow[0, :, 1, ::STRIDE0, ::STRIDE0] * dtv
        else:
            fw = bflow[0, :, 0, ::STRIDE0, ::STRIDE0] * (-dtv)
            fh = bflow[0, :, 1, ::STRIDE0, ::STRIDE0] * (-dtv)
        c0h = jnp.round(qh[None, :, None] + fh).astype(jnp.int32)
        c0w = jnp.round(qh[None, None, :] + fw).astype(jnp.int32)
        u = jnp.clip(c0h[..., None] + owi[None, None, None, :], 0, H - 1)
        v = jnp.clip(c0w[..., None] + owj[None, None, None, :], 0, W - 1)
        cbs.append(u * WP + v)
    cb = jnp.stack(cbs, axis=1)
    cb_r = jnp.transpose(cb, (2, 0, 1, 3, 4)).reshape(NQ * T * NDT, NQ * NGL)

    eoff = jnp.asarray(_EOFF_PAD)

    mesh = plsc.VectorSubcoreMesh(core_axis_name="c", subcore_axis_name="s")
    run = pl.kernel(
        _sc_body,
        out_type=jax.ShapeDtypeStruct((_NW, 16), jnp.float32),
        mesh=mesh,
        compiler_params=pltpu.CompilerParams(needs_layout_passes=False),
        scratch_types=[
            pltpu.VMEM((FRAME_PAD,), jnp.float32),
            pltpu.VMEM((FRAME_PAD,), jnp.float32),
            pltpu.VMEM((NQ, NEG * 16), jnp.float32),
            pltpu.VMEM((NEG * 16 * NQ,), jnp.float32),
            pltpu.VMEM((NQ, 16), jnp.float32),
            pltpu.VMEM((NQ * 16,), jnp.float32),
            pltpu.VMEM((NQ * NGL,), jnp.int32),
            pltpu.VMEM((NEG * 16,), jnp.int32),
            pltpu.VMEM((16,), jnp.float32),
            pltpu.SemaphoreType.DMA,
        ],
    )
    partials = run(vp, cb_r, eoff)
    return jnp.sum(partials) / jnp.float32(T * NQ * NQ * K)


# speedup vs baseline: 10.0080x; 1.0849x over previous
import numpy as np
import jax
import jax.numpy as jnp
from jax import lax
from jax.experimental import pallas as pl
from jax.experimental.pallas import tpu as pltpu
from jax.experimental.pallas import tpu_sc as plsc

WS = 9
PS = 7
K = 10
STRIDE0 = 4
T, C, H, W = 5, 3, 128, 128
PAD = PS // 2
HP = H + 2 * PAD
WP = 136
FRAME_PAD = C * HP * WP
NQ = H // STRIDE0
D = C * PS * PS
NWIN = WS * WS
NDT = 3
NG = 6
NGL = NG * 16
NEG = 10
DCOL = NDT * NGL
BIG = 1e30

_EOFF = [c * (HP * WP) + i * WP + j
         for c in range(C) for i in range(PS) for j in range(PS)]
_EOFF_PAD = np.array(_EOFF + [0] * (NEG * 16 - D), dtype=np.int32)

_OWI = np.array([n // WS - WS // 2 if n < NWIN else 0 for n in range(NGL)],
                dtype=np.int32)
_OWJ = np.array([n % WS - WS // 2 if n < NWIN else 0 for n in range(NGL)],
                dtype=np.int32)

_NC = 2
_NS = 16
_NW = _NC * _NS


def _sc_body(vp_h, cbase_h, eoff_h, out_h,
             fba, fbb, qpat, qpatT, rbuf, lbuf, cbbuf, eofb, accb, dsem):
    wid = lax.axis_index("s") * _NC + lax.axis_index("c")
    pltpu.sync_copy(eoff_h, eofb)
    accb[...] = jnp.zeros((16,), jnp.float32)
    pltpu.sync_copy(vp_h.at[0], fba)

    ybase = wid * STRIDE0 * WP

    def compute_dt(t, dt, fbuf):
        pltpu.sync_copy(cbase_h.at[(wid * T + t) * NDT + dt], cbbuf)
        def q_body(q, c2):
            def g_body(g, prev):
                gb = g * 16
                cb = cbbuf[pl.ds(q * NGL + gb, 16)]
                cbj = [cb + j for j in range(PS)]
                accs = [jnp.zeros((16,), jnp.float32) for _ in range(8)]
                qv = None
                for e in range(D):
                    if e % 16 == 0:
                        qv = qpat[q, pl.ds(e, 16)]
                    ci, j = divmod(e, PS)
                    rb = (ci // PS) * (HP * WP) + (ci % PS) * WP
                    fv = fbuf.at[pl.ds(rb, FRAME_PAD - rb)]
                    pix = plsc.load_gather(fv, [cbj[j]])
                    d = qv[e % 16] - pix
                    accs[e % 8] = accs[e % 8] + d * d
                acc = ((accs[0] + accs[1]) + (accs[2] + accs[3])) + (
                    (accs[4] + accs[5]) + (accs[6] + accs[7]))
                r = rbuf[q, :]
                rbuf[q, :] = lax.sort(
                    jnp.minimum(r, lax.rev(lax.sort(prev), (0,))))
                return acc
            prev0 = jnp.full((16,), BIG, jnp.float32)
            last = lax.fori_loop(0, NG - 1, g_body, prev0)
            r = rbuf[q, :]
            rbuf[q, :] = lax.sort(
                jnp.minimum(r, lax.rev(lax.sort(last), (0,))))
            return c2
        lax.fori_loop(0, NQ, q_body, 0)

        def lo_body(qg, c2):
            lane = lax.broadcasted_iota(jnp.int32, (16,), 0)
            qlane = lane + qg * 16
            cb = plsc.load_gather(cbbuf, [qlane * NGL + (NWIN - 1)])

            def le_body(eg, accs):
                ev = eofb[pl.ds(eg * 16, 16)]
                base = eg * 16 * NQ + qg * 16
                nacc = list(accs)
                for l in range(16):
                    pix = plsc.load_gather(fbuf, [cb + ev[l]])
                    qvv = qpatT[pl.ds(base + l * NQ, 16)]
                    d = qvv - pix
                    nacc[l % 8] = nacc[l % 8] + d * d
                return tuple(nacc)
            accs = lax.fori_loop(
                0, NEG - 1, le_body,
                tuple(jnp.zeros((16,), jnp.float32) for _ in range(8)))
            ev = eofb[pl.ds((NEG - 1) * 16, 16)]
            base = (NEG - 1) * 16 * NQ + qg * 16
            accs = list(accs)
            for l in range(D - (NEG - 1) * 16):
                pix = plsc.load_gather(fbuf, [cb + ev[l]])
                qvv = qpatT[pl.ds(base + l * NQ, 16)]
                d = qvv - pix
                accs[l % 8] = accs[l % 8] + d * d
            acc = ((accs[0] + accs[1]) + (accs[2] + accs[3])) + (
                (accs[4] + accs[5]) + (accs[6] + accs[7]))
            plsc.store_scatter(lbuf, [qlane * 16 + dt], acc)
            return c2
        lax.fori_loop(0, 2, lo_body, 0)

    def t_body(t, tcarry):

        def qg_body(q, c):
            qb = ybase + q * STRIDE0

            def g_body(g, c2):
                ev = eofb[pl.ds(g * 16, 16)]
                pix = plsc.load_gather(fba, [ev + qb])
                qpat[q, pl.ds(g * 16, 16)] = pix
                lane = lax.broadcasted_iota(jnp.int32, (16,), 0)
                plsc.store_scatter(qpatT, [(lane + g * 16) * NQ + q], pix)
                return c2
            return lax.fori_loop(0, NEG, g_body, c)
        lax.fori_loop(0, NQ, qg_body, 0)

        def ri_body(q, c):
            rbuf[q, :] = jnp.full((16,), BIG, jnp.float32)
            lbuf[pl.ds(q * 16, 16)] = jnp.full((16,), BIG, jnp.float32)
            return c
        lax.fori_loop(0, NQ, ri_body, 0)

        cpB = pltpu.async_copy(vp_h.at[jnp.clip(t - 1, 0, T - 1)], fbb, dsem)
        compute_dt(t, 1, fba)
        cpB.wait()
        cpA = pltpu.async_copy(vp_h.at[jnp.clip(t + 1, 0, T - 1)], fba, dsem)
        compute_dt(t, 0, fbb)
        cpA.wait()
        compute_dt(t, 2, fba)

        def qt_body(q, c):
            r = rbuf[q, :]
            lv = lbuf[pl.ds(q * 16, 16)]
            r = lax.sort(jnp.minimum(r, lax.rev(lax.sort(lv), (0,))))
            lane = lax.broadcasted_iota(jnp.int32, (16,), 0)
            accb[...] = accb[...] + jnp.where(lane < K, r, jnp.float32(0.0))
            return c
        lax.fori_loop(0, NQ, qt_body, 0)
        return tcarry

    lax.fori_loop(0, T, t_body, 0)
    pltpu.sync_copy(accb, out_h.at[wid])


@jax.jit
def kernel(noisy, deno, fflow, bflow):
    del deno
    vid = noisy[0]
    vp = jnp.pad(vid, ((0, 0), (0, 0), (PAD, PAD), (PAD, WP - W - PAD)),
                 mode='edge')
    vp = vp.reshape(T, FRAME_PAD)

    qh = jnp.arange(0, H, STRIDE0, dtype=jnp.float32)
    owi = jnp.asarray(_OWI)
    owj = jnp.asarray(_OWJ)
    cbs = []
    for dtv in (-1, 0, 1):
        if dtv == 0:
            fh = jnp.zeros((T, NQ, NQ), jnp.float32)
            fw = fh
        elif dtv > 0:
            fw = fflow[0, :, 0, ::STRIDE0, ::STRIDE0] * dtv
            fh = fflow[0, :, 1, ::STRIDE0, ::STRIDE0] * dtv
        else:
            fw = bflow[0, :, 0, ::STRIDE0, ::STRIDE0] * (-dtv)
            fh = bflow[0, :, 1, ::STRIDE0, ::STRIDE0] * (-dtv)
        c0h = jnp.round(qh[None, :, None] + fh).astype(jnp.int32)
        c0w = jnp.round(qh[None, None, :] + fw).astype(jnp.int32)
        u = jnp.clip(c0h[..., None] + owi[None, None, None, :], 0, H - 1)
        v = jnp.clip(c0w[..., None] + owj[None, None, None, :], 0, W - 1)
        cbs.append(u * WP + v)
    cb = jnp.stack(cbs, axis=1)
    cb_r = jnp.transpose(cb, (2, 0, 1, 3, 4)).reshape(NQ * T * NDT, NQ * NGL)

    eoff = jnp.asarray(_EOFF_PAD)

    mesh = plsc.VectorSubcoreMesh(core_axis_name="c", subcore_axis_name="s")
    run = pl.kernel(
        _sc_body,
        out_type=jax.ShapeDtypeStruct((_NW, 16), jnp.float32),
        mesh=mesh,
        compiler_params=pltpu.CompilerParams(needs_layout_passes=False),
        scratch_types=[
            pltpu.VMEM((FRAME_PAD,), jnp.float32),
            pltpu.VMEM((FRAME_PAD,), jnp.float32),
            pltpu.VMEM((NQ, NEG * 16), jnp.float32),
            pltpu.VMEM((NEG * 16 * NQ,), jnp.float32),
            pltpu.VMEM((NQ, 16), jnp.float32),
            pltpu.VMEM((NQ * 16,), jnp.float32),
            pltpu.VMEM((NQ * NGL,), jnp.int32),
            pltpu.VMEM((NEG * 16,), jnp.int32),
            pltpu.VMEM((16,), jnp.float32),
            pltpu.SemaphoreType.DMA,
        ],
    )
    partials = run(vp, cb_r, eoff)
    return jnp.sum(partials) / jnp.float32(T * NQ * NQ * K)
